# Initial kernel scaffold; baseline (speedup 1.0000x reference)
import jax
import jax.numpy as jnp
from jax.experimental import pallas as pl


def kernel(X, W0, b0, W1, b1, W2, b2, W3, b3, vertex, edges):
    raise NotImplementedError("write your pallas kernel here")


# speedup vs baseline: 4.7998x; 4.7998x over previous
import functools

import jax
import jax.numpy as jnp
from jax import lax
from jax.experimental import pallas as pl
from jax.experimental.pallas import tpu as pltpu
from jax.experimental.pallas import tpu_sc as plsc

N_V = 10000
E_E = 40000
NNZ = 320000
D = 128
L0, L1 = 13334, 26667
EP = 40064
NP = 10112
DEG_SCALE = 0.25
SLOPE = 0.2

NC, NS = 2, 16
G = 128
NG = NNZ // G


def _mm_body(x_ref, w_ref, b_ref, o_ref):
    t = pl.program_id(0)
    x = x_ref[...]
    w = w_ref[0]
    y = lax.dot_general(x, w, (((1,), (1,)), ((), ())),
                        preferred_element_type=jnp.float32)
    y = y + b_ref[0]
    scale = jnp.where(t == 0, 1.0, DEG_SCALE).astype(jnp.float32)
    o_ref[...] = y * scale


def _matmuls(X, W4, b4):
    bn = 1000
    nb = N_V // bn
    return pl.pallas_call(
        _mm_body,
        grid=(4, nb),
        in_specs=[
            pl.BlockSpec((bn, D), lambda t, i: (i, 0)),
            pl.BlockSpec((1, D, D), lambda t, i: (t, 0, 0)),
            pl.BlockSpec((1, 1, D), lambda t, i: (t, 0, 0)),
        ],
        out_specs=pl.BlockSpec((bn, D), lambda t, i: (t * nb + i, 0)),
        out_shape=jax.ShapeDtypeStruct((4 * N_V, D), jnp.float32),
    )(X, W4, b4)


def _p1_body(ys32, vertex, edges, zeros_e32, zeros_e8, ones_g8,
             sums4, cnt8, vbuf, ebuf, gbuf, rows, onesb, acc, cacc, sem):
    cid = lax.axis_index("c")
    sid = lax.axis_index("s")
    er = EP // NS
    estripe = pl.ds(sid * er, er)
    pltpu.sync_copy(ones_g8, onesb)

    base_g = sid * (NG // NS) + jnp.minimum(sid, NG % NS)
    ngs = jnp.where(sid < NG % NS, NG // NS + 1, NG // NS)

    for p in range(2):
        kslice = cid * 2 + p
        pltpu.sync_copy(zeros_e32.at[estripe], acc.at[estripe])
        if p == 0:
            @pl.when(cid == 1)
            def _():
                pltpu.sync_copy(zeros_e8.at[estripe], cacc.at[estripe])
        plsc.subcore_barrier()

        def group_body(gi, carry):
            off = (base_g + gi) * G
            pltpu.sync_copy(vertex.at[pl.ds(off, G)], vbuf)
            pltpu.sync_copy(edges.at[pl.ds(off, G)], ebuf)
            for j in range(G // 16):
                sl = pl.ds(j * 16, 16)
                e = ebuf[sl]
                v = vbuf[sl]
                t = (jnp.where(e >= L0, 1, 0) + jnp.where(e >= L1, 1, 0))
                gbuf[sl] = ((t + 1) * N_V + v) * 4 + kslice
            pltpu.async_copy(ys32.at[gbuf], rows, sem).wait()
            pltpu.sync_copy(rows, acc.at[ebuf], add=True)
            if p == 0:
                @pl.when(cid == 1)
                def _():
                    pltpu.sync_copy(onesb, cacc.at[ebuf], add=True)
            return carry

        lax.fori_loop(0, ngs, group_body, 0)
        plsc.subcore_barrier()
        pltpu.sync_copy(acc.at[estripe], sums4.at[kslice, estripe])
        if p == 0:
            @pl.when(cid == 1)
            def _():
                pltpu.sync_copy(cacc.at[estripe], cnt8.at[estripe])


def _phase1(ys32, vertex, edges):
    mesh = plsc.VectorSubcoreMesh(core_axis_name="c", subcore_axis_name="s")
    fn = functools.partial(
        pl.kernel,
        mesh=mesh,
        compiler_params=pltpu.CompilerParams(use_tc_tiling_on_sc=False),
        out_type=[
            jax.ShapeDtypeStruct((4, EP, 32), jnp.float32),
            jax.ShapeDtypeStruct((EP, 8), jnp.float32),
        ],
        scratch_types=[
            pltpu.VMEM((G,), jnp.int32),
            pltpu.VMEM((G,), jnp.int32),
            pltpu.VMEM((G,), jnp.int32),
            pltpu.VMEM((G, 32), jnp.float32),
            pltpu.VMEM((G, 8), jnp.float32),
            pltpu.VMEM_SHARED((EP, 32), jnp.float32),
            pltpu.VMEM_SHARED((EP, 8), jnp.float32),
            pltpu.SemaphoreType.DMA,
        ],
    )(_p1_body)
    zeros_e32 = jnp.zeros((EP, 32), jnp.float32)
    zeros_e8 = jnp.zeros((EP, 8), jnp.float32)
    ones_g8 = jnp.ones((G, 8), jnp.float32)
    return fn(ys32, vertex, edges, zeros_e32, zeros_e8, ones_g8)


def _div_body(s0_ref, s1_ref, s2_ref, s3_ref, c_ref, o_ref):
    cnt = c_ref[...][:, 0:1]
    rec = 1.0 / jnp.maximum(cnt, 1.0)
    for k, sref in enumerate((s0_ref, s1_ref, s2_ref, s3_ref)):
        o_ref[:, k * 32:(k + 1) * 32] = sref[0] * rec


def _divide(sums4, cnt8):
    be = 1000
    nb = E_E // be

    def sspec(k):
        return pl.BlockSpec((1, be, 32), lambda i, k=k: (k, i, 0))

    return pl.pallas_call(
        _div_body,
        grid=(nb,),
        in_specs=[sspec(0), sspec(1), sspec(2), sspec(3),
                  pl.BlockSpec((be, 8), lambda i: (i, 0))],
        out_specs=pl.BlockSpec((be, D), lambda i: (i, 0)),
        out_shape=jax.ShapeDtypeStruct((E_E, D), jnp.float32),
    )(sums4, sums4, sums4, sums4, cnt8)


def _p2_body(xe, vertex, edges, zeros_n, pv, vbuf, ebuf, rows, acc, sem):
    cid = lax.axis_index("c")
    sid = lax.axis_index("s")
    nr = NP // NS
    nstripe = pl.ds(sid * nr, nr)
    pltpu.sync_copy(zeros_n.at[nstripe], acc.at[nstripe])
    plsc.subcore_barrier()

    wid = sid * NC + cid
    nw = NC * NS
    base_g = wid * (NG // nw) + jnp.minimum(wid, NG % nw)
    ngs = jnp.where(wid < NG % nw, NG // nw + 1, NG // nw)

    def group_body(gi, carry):
        off = (base_g + gi) * G
        pltpu.sync_copy(vertex.at[pl.ds(off, G)], vbuf)
        pltpu.sync_copy(edges.at[pl.ds(off, G)], ebuf)
        pltpu.async_copy(xe.at[ebuf], rows, sem).wait()
        pltpu.sync_copy(rows, acc.at[vbuf], add=True)
        return carry

    lax.fori_loop(0, ngs, group_body, 0)
    plsc.subcore_barrier()
    pltpu.sync_copy(acc.at[nstripe], pv.at[cid, nstripe])


def _phase2(xe, vertex, edges):
    mesh = plsc.VectorSubcoreMesh(core_axis_name="c", subcore_axis_name="s")
    fn = functools.partial(
        pl.kernel,
        mesh=mesh,
        compiler_params=pltpu.CompilerParams(use_tc_tiling_on_sc=False),
        out_type=jax.ShapeDtypeStruct((NC, NP, D), jnp.float32),
        scratch_types=[
            pltpu.VMEM((G,), jnp.int32),
            pltpu.VMEM((G,), jnp.int32),
            pltpu.VMEM((G, D), jnp.float32),
            pltpu.VMEM_SHARED((NP, D), jnp.float32),
            pltpu.SemaphoreType.DMA,
        ],
    )(_p2_body)
    zeros_n = jnp.zeros((NP, D), jnp.float32)
    return fn(xe, vertex, edges, zeros_n)


def _fin_body(pv_ref, x0_ref, o_ref):
    x = pv_ref[0] + pv_ref[1] + x0_ref[...]
    norm = jnp.sqrt(jnp.sum(x * x, axis=1, keepdims=True))
    scale = jnp.where(norm > 0, 1.0 / norm, 0.0)
    y = x * scale
    o_ref[...] = jnp.where(y > 0, y, SLOPE * y)


def _finalize(pv, ys):
    bn = 1000
    nb = N_V // bn
    return pl.pallas_call(
        _fin_body,
        grid=(nb,),
        in_specs=[
            pl.BlockSpec((NC, bn, D), lambda i: (0, i, 0)),
            pl.BlockSpec((bn, D), lambda i: (i, 0)),
        ],
        out_specs=pl.BlockSpec((bn, D), lambda i: (i, 0)),
        out_shape=jax.ShapeDtypeStruct((N_V, D), jnp.float32),
    )(pv, ys)


def kernel(X, W0, b0, W1, b1, W2, b2, W3, b3, vertex, edges):
    W4 = jnp.stack([W0, W1, W2, W3])
    b4 = jnp.stack([b0, b1, b2, b3]).reshape(4, 1, D)
    ys = _matmuls(X, W4, b4)
    ys32 = ys.reshape(16 * N_V, 32)
    sums4, cnt8 = _phase1(ys32, vertex, edges)
    xe = _divide(sums4, cnt8)
    pv = _phase2(xe, vertex, edges)
    return _finalize(pv, ys)


# speedup vs baseline: 10.0492x; 2.0937x over previous
import functools

import jax
import jax.numpy as jnp
from jax import lax
from jax.experimental import pallas as pl
from jax.experimental.pallas import tpu as pltpu
from jax.experimental.pallas import tpu_sc as plsc

N_V = 10000
E_E = 40000
NNZ = 320000
D = 128
L0, L1 = 13334, 26667
EP = 40064
NP = 10112
DEG_SCALE = 0.25
SLOPE = 0.2

NC, NS = 2, 16
LROW = 128
NROW = NNZ // LROW
Q1 = 2
Q2 = 1


def _mm_body(x_ref, w_ref, b_ref, o_ref):
    t = pl.program_id(0)
    x = x_ref[...]
    w = w_ref[0]
    y = lax.dot_general(x, w, (((1,), (1,)), ((), ())),
                        preferred_element_type=jnp.float32)
    y = y + b_ref[0]
    scale = jnp.where(t == 0, 1.0, DEG_SCALE).astype(jnp.float32)
    o_ref[...] = y * scale


def _matmuls(X, W4, b4):
    bn = 1000
    nb = N_V // bn
    return pl.pallas_call(
        _mm_body,
        grid=(4, nb),
        in_specs=[
            pl.BlockSpec((bn, D), lambda t, i: (i, 0)),
            pl.BlockSpec((1, D, D), lambda t, i: (t, 0, 0)),
            pl.BlockSpec((1, 1, D), lambda t, i: (t, 0, 0)),
        ],
        out_specs=pl.BlockSpec((bn, D), lambda t, i: (t * nb + i, 0)),
        out_shape=jax.ShapeDtypeStruct((4 * N_V, D), jnp.float32),
    )(X, W4, b4)


def _p1_body(ys32, vtx2, edg2, zeros_e32, zeros_e8, ones_g8,
             sums4, cnt8,
             vbufA, ebufA, gbufA, rowsA, vbufB, ebufB, gbufB, rowsB,
             onesb, acc, cacc, isem, gsem, ssem):
    cid = lax.axis_index("c")
    sid = lax.axis_index("s")
    er = EP // NS
    estripe = pl.ds(sid * er, er)
    pltpu.sync_copy(ones_g8, onesb)

    ngrp = NROW // Q1
    ngd, ngm = ngrp // NS, ngrp % NS
    base_g = sid * ngd + jnp.minimum(sid, ngm)
    ngs = jnp.where(sid < ngm, ngd + 1, ngd)
    bufs = ((vbufA, ebufA, gbufA, rowsA), (vbufB, ebufB, gbufB, rowsB))

    def issue_idx(g, b):
        vb, eb = bufs[b][0], bufs[b][1]
        off = (base_g + g) * Q1
        pltpu.async_copy(vtx2.at[pl.ds(off, Q1)], vb, isem)
        pltpu.async_copy(edg2.at[pl.ds(off, Q1)], eb, isem)

    def wait_idx(b):
        vb, eb = bufs[b][0], bufs[b][1]
        pltpu.make_async_copy(vtx2.at[pl.ds(0, Q1)], vb, isem).wait()
        pltpu.make_async_copy(edg2.at[pl.ds(0, Q1)], eb, isem).wait()

    def drain_scatter(p):
        for q in range(Q1):
            pltpu.make_async_copy(
                rowsA.at[pl.ds(q * LROW, LROW)], acc.at[ebufA.at[q]], ssem
            ).wait()
            if p == 0:
                pltpu.make_async_copy(onesb, cacc.at[ebufA.at[q]], ssem).wait()

    def step(g, b, p, kslice, first):
        vb, eb, gb, rw = bufs[b]
        wait_idx(b)
        for q in range(Q1):
            for j in range(LROW // 16):
                sl = pl.ds(j * 16, 16)
                e = eb[q, sl]
                v = vb[q, sl]
                t = jnp.where(e >= L0, 1, 0) + jnp.where(e >= L1, 1, 0)
                gb[q, sl] = ((t + 1) * N_V + v) * 4 + kslice
        for q in range(Q1):
            pltpu.async_copy(ys32.at[gb.at[q]],
                             rw.at[pl.ds(q * LROW, LROW)], gsem)
        if not first:
            drain_scatter(p)

        @pl.when(g + 1 < ngs)
        def _():
            issue_idx(g + 1, 1 - b)

        for q in range(Q1):
            pltpu.make_async_copy(ys32.at[gb.at[q]],
                                  rw.at[pl.ds(q * LROW, LROW)], gsem).wait()
        for q in range(Q1):
            pltpu.async_copy(rw.at[pl.ds(q * LROW, LROW)],
                             acc.at[eb.at[q]], ssem, add=True)
            if p == 0:
                pltpu.async_copy(onesb, cacc.at[eb.at[q]], ssem, add=True)

    for p in range(2):
        kslice = cid * 2 + p
        pltpu.sync_copy(zeros_e32.at[estripe], acc.at[estripe])
        if p == 0:
            pltpu.sync_copy(zeros_e8.at[estripe], cacc.at[estripe])
        plsc.subcore_barrier()

        issue_idx(0, 0)
        step(0, 0, p, kslice, True)

        def pair_body(g2, carry):
            ga = 1 + 2 * g2

            @pl.when(ga < ngs)
            def _():
                step(ga, 1, p, kslice, False)

            gb_ = 2 + 2 * g2

            @pl.when(gb_ < ngs)
            def _():
                step(gb_, 0, p, kslice, False)

            return carry

        lax.fori_loop(0, (ngd + 1) // 2 + 1, pair_body, 0)
        drain_scatter(p)
        plsc.subcore_barrier()
        pltpu.sync_copy(acc.at[estripe], sums4.at[kslice, estripe])
        if p == 0:
            @pl.when(cid == 1)
            def _():
                pltpu.sync_copy(cacc.at[estripe], cnt8.at[estripe])


def _phase1(ys32, vtx2, edg2):
    mesh = plsc.VectorSubcoreMesh(core_axis_name="c", subcore_axis_name="s")
    fn = functools.partial(
        pl.kernel,
        mesh=mesh,
        compiler_params=pltpu.CompilerParams(use_tc_tiling_on_sc=False),
        out_type=[
            jax.ShapeDtypeStruct((4, EP, 32), jnp.float32),
            jax.ShapeDtypeStruct((EP, 8), jnp.float32),
        ],
        scratch_types=(
            [pltpu.VMEM((Q1, LROW), jnp.int32)] * 3
            + [pltpu.VMEM((Q1 * LROW, 32), jnp.float32)]
            + [pltpu.VMEM((Q1, LROW), jnp.int32)] * 3
            + [pltpu.VMEM((Q1 * LROW, 32), jnp.float32)]
            + [
                pltpu.VMEM((LROW, 8), jnp.float32),
                pltpu.VMEM_SHARED((EP, 32), jnp.float32),
                pltpu.VMEM_SHARED((EP, 8), jnp.float32),
                pltpu.SemaphoreType.DMA,
                pltpu.SemaphoreType.DMA,
                pltpu.SemaphoreType.DMA,
            ]
        ),
    )(_p1_body)
    zeros_e32 = jnp.zeros((EP, 32), jnp.float32)
    zeros_e8 = jnp.zeros((EP, 8), jnp.float32)
    ones_g8 = jnp.ones((LROW, 8), jnp.float32)
    return fn(ys32, vtx2, edg2, zeros_e32, zeros_e8, ones_g8)


def _div_body(s0_ref, s1_ref, s2_ref, s3_ref, c_ref, o_ref):
    cnt = c_ref[...][:, 0:1]
    rec = 1.0 / jnp.maximum(cnt, 1.0)
    for k, sref in enumerate((s0_ref, s1_ref, s2_ref, s3_ref)):
        o_ref[:, k * 32:(k + 1) * 32] = sref[0] * rec


def _divide(sums4, cnt8):
    be = 1000
    nb = E_E // be

    def sspec(k):
        return pl.BlockSpec((1, be, 32), lambda i, k=k: (k, i, 0))

    return pl.pallas_call(
        _div_body,
        grid=(nb,),
        in_specs=[sspec(0), sspec(1), sspec(2), sspec(3),
                  pl.BlockSpec((be, 8), lambda i: (i, 0))],
        out_specs=pl.BlockSpec((be, D), lambda i: (i, 0)),
        out_shape=jax.ShapeDtypeStruct((E_E, D), jnp.float32),
    )(sums4, sums4, sums4, sums4, cnt8)


def _p2_body(xe, vtx2, edg2, zeros_n, pv,
             vbufA, ebufA, rowsA, vbufB, ebufB, rowsB, acc, isem, gsem, ssem):
    cid = lax.axis_index("c")
    sid = lax.axis_index("s")
    nr = NP // NS
    nstripe = pl.ds(sid * nr, nr)
    pltpu.sync_copy(zeros_n.at[nstripe], acc.at[nstripe])
    plsc.subcore_barrier()

    wid = sid * NC + cid
    nw = NC * NS
    ngrp = NROW // Q2
    ngd, ngm = ngrp // nw, ngrp % nw
    base_g = wid * ngd + jnp.minimum(wid, ngm)
    ngs = jnp.where(wid < ngm, ngd + 1, ngd)
    bufs = ((vbufA, ebufA, rowsA), (vbufB, ebufB, rowsB))

    def issue_idx(g, b):
        vb, eb = bufs[b][0], bufs[b][1]
        off = (base_g + g) * Q2
        pltpu.async_copy(vtx2.at[pl.ds(off, Q2)], vb, isem)
        pltpu.async_copy(edg2.at[pl.ds(off, Q2)], eb, isem)

    def wait_idx(b):
        vb, eb = bufs[b][0], bufs[b][1]
        pltpu.make_async_copy(vtx2.at[pl.ds(0, Q2)], vb, isem).wait()
        pltpu.make_async_copy(edg2.at[pl.ds(0, Q2)], eb, isem).wait()

    def drain_scatter():
        for q in range(Q2):
            pltpu.make_async_copy(
                rowsA.at[pl.ds(q * LROW, LROW)], acc.at[vbufA.at[q]], ssem
            ).wait()

    def step(g, b, first):
        vb, eb, rw = bufs[b]
        wait_idx(b)
        for q in range(Q2):
            pltpu.async_copy(xe.at[eb.at[q]],
                             rw.at[pl.ds(q * LROW, LROW)], gsem)
        if not first:
            drain_scatter()

        @pl.when(g + 1 < ngs)
        def _():
            issue_idx(g + 1, 1 - b)

        for q in range(Q2):
            pltpu.make_async_copy(xe.at[eb.at[q]],
                                  rw.at[pl.ds(q * LROW, LROW)], gsem).wait()
        for q in range(Q2):
            pltpu.async_copy(rw.at[pl.ds(q * LROW, LROW)],
                             acc.at[vb.at[q]], ssem, add=True)

    issue_idx(0, 0)
    step(0, 0, True)

    def pair_body(g2, carry):
        ga = 1 + 2 * g2

        @pl.when(ga < ngs)
        def _():
            step(ga, 1, False)

        gb_ = 2 + 2 * g2

        @pl.when(gb_ < ngs)
        def _():
            step(gb_, 0, False)

        return carry

    lax.fori_loop(0, (ngd + 1) // 2 + 1, pair_body, 0)
    drain_scatter()
    plsc.subcore_barrier()
    pltpu.sync_copy(acc.at[nstripe], pv.at[cid, nstripe])


def _phase2(xe, vtx2, edg2):
    mesh = plsc.VectorSubcoreMesh(core_axis_name="c", subcore_axis_name="s")
    fn = functools.partial(
        pl.kernel,
        mesh=mesh,
        compiler_params=pltpu.CompilerParams(use_tc_tiling_on_sc=False),
        out_type=jax.ShapeDtypeStruct((NC, NP, D), jnp.float32),
        scratch_types=(
            [pltpu.VMEM((Q2, LROW), jnp.int32)] * 2
            + [pltpu.VMEM((Q2 * LROW, D), jnp.float32)]
            + [pltpu.VMEM((Q2, LROW), jnp.int32)] * 2
            + [pltpu.VMEM((Q2 * LROW, D), jnp.float32)]
            + [
                pltpu.VMEM_SHARED((NP, D), jnp.float32),
                pltpu.SemaphoreType.DMA,
                pltpu.SemaphoreType.DMA,
                pltpu.SemaphoreType.DMA,
            ]
        ),
    )(_p2_body)
    zeros_n = jnp.zeros((NP, D), jnp.float32)
    return fn(xe, vtx2, edg2, zeros_n)


def _fin_body(pv_ref, x0_ref, o_ref):
    x = pv_ref[0] + pv_ref[1] + x0_ref[...]
    norm = jnp.sqrt(jnp.sum(x * x, axis=1, keepdims=True))
    scale = jnp.where(norm > 0, 1.0 / norm, 0.0)
    y = x * scale
    o_ref[...] = jnp.where(y > 0, y, SLOPE * y)


def _finalize(pv, ys):
    bn = 1000
    nb = N_V // bn
    return pl.pallas_call(
        _fin_body,
        grid=(nb,),
        in_specs=[
            pl.BlockSpec((NC, bn, D), lambda i: (0, i, 0)),
            pl.BlockSpec((bn, D), lambda i: (i, 0)),
        ],
        out_specs=pl.BlockSpec((bn, D), lambda i: (i, 0)),
        out_shape=jax.ShapeDtypeStruct((N_V, D), jnp.float32),
    )(pv, ys)


def kernel(X, W0, b0, W1, b1, W2, b2, W3, b3, vertex, edges):
    W4 = jnp.stack([W0, W1, W2, W3])
    b4 = jnp.stack([b0, b1, b2, b3]).reshape(4, 1, D)
    vtx2 = vertex.reshape(NROW, LROW)
    edg2 = edges.reshape(NROW, LROW)
    ys = _matmuls(X, W4, b4)
    ys32 = ys.reshape(16 * N_V, 32)
    sums4, cnt8 = _phase1(ys32, vtx2, edg2)
    xe = _divide(sums4, cnt8)
    pv = _phase2(xe, vtx2, edg2)
    return _finalize(pv, ys)
